# baseline (speedup 1.0000x reference)
import functools

import jax
import jax.numpy as jnp
from jax import lax
from jax.experimental import pallas as pl
from jax.experimental.pallas import tpu as pltpu
from jax.experimental.pallas import tpu_sc as plsc

N = 10000
E = 320000
IN = 128
HID = 1000
HIDP = 1024
OUT = 256
ATT_H = 128

NP = 10240

NC = 2
NS = 16
NW = NC * NS
EPW = E // NW
HB = 64
EB = 2 * HB
NB2 = 80
EPAD = NB2 * EB - EPW
RPS = NP // NS
L = 16

DL = 8

RB = 400
GN = N // RB


@functools.cache
def _mesh():
    return plsc.VectorSubcoreMesh(
        core_axis_name="c", subcore_axis_name="s",
        num_cores=NC, num_subcores=NS)


def _deg_body(e0_hbm, e1_hbm, degp_hbm,
              d0, d1, d2, d3, shared, idx_v, buf, accum):
    c = lax.axis_index("c")
    s = lax.axis_index("s")
    w = c * NS + s
    r0 = s * RPS
    degs = [d0, d1, d2, d3]
    zeros16 = jnp.zeros((L,), jnp.float32)
    ones16 = jnp.ones((L,), jnp.float32)

    def zbody(t, carry):
        for j in range(4):
            degs[j][pl.ds(t * L, L)] = zeros16
        return carry

    lax.fori_loop(0, NP // L, zbody, 0)

    for e_hbm, j0 in ((e0_hbm, 0), (e1_hbm, 2)):
        def body(b, carry, e_hbm=e_hbm, j0=j0):
            pltpu.sync_copy(e_hbm.at[w, b], idx_v)
            for r in range(4):
                dj = degs[j0 + (r // 2)]
                for i in range(HB // L):
                    idx = idx_v[r, pl.ds(i * L, L)]
                    plsc.addupdate_scatter(dj, [idx], ones16)
            return carry

        lax.fori_loop(0, NB2, body, 0)

    for j in range(4):
        pltpu.sync_copy(degs[j], shared.at[j, s])
    plsc.subcore_barrier()

    for j in range(4):
        pltpu.sync_copy(shared.at[j, :, pl.ds(r0, RPS)], buf)

        def abody(t, carry):
            acc = buf[0, pl.ds(t * L, L)]
            for k in range(1, NS):
                acc = acc + buf[k, pl.ds(t * L, L)]
            accum[pl.ds(t * L, L)] = acc
            return carry

        lax.fori_loop(0, RPS // L, abody, 0)
        pltpu.sync_copy(accum, degp_hbm.at[c, j, pl.ds(r0, RPS)])


@functools.cache
def _deg_kernel():
    return pl.kernel(
        _deg_body,
        out_type=jax.ShapeDtypeStruct((NC, 4, NP), jnp.float32),
        mesh=_mesh(),
        scratch_types=[
            pltpu.VMEM((NP,), jnp.float32),
            pltpu.VMEM((NP,), jnp.float32),
            pltpu.VMEM((NP,), jnp.float32),
            pltpu.VMEM((NP,), jnp.float32),
            pltpu.VMEM_SHARED((4, NS, NP), jnp.float32),
            pltpu.VMEM((4, HB), jnp.int32),
            pltpu.VMEM((NS, RPS), jnp.float32),
            pltpu.VMEM((RPS,), jnp.float32),
        ],
        compiler_params=pltpu.CompilerParams(needs_layout_passes=False),
    )


def _agg_phase(table_hbm, eidx_hbm, zeros_hbm, out_hbm, j, acc,
               idx_v, rows1, rows2, gs1, gs2, ss1, ss2, c, w, r0):
    pltpu.sync_copy(zeros_hbm.at[pl.ds(r0, RPS), :],
                    acc.at[pl.ds(r0, RPS), :])
    plsc.subcore_barrier()

    def body(b, carry):
        pltpu.sync_copy(eidx_hbm.at[w, b], idx_v)
        g1 = pltpu.async_copy(table_hbm.at[idx_v.at[0]], rows1, gs1)
        g2 = pltpu.async_copy(table_hbm.at[idx_v.at[1]], rows2, gs2)
        g1.wait()
        pltpu.sync_copy(rows1, acc.at[idx_v.at[2]], add=True)
        g2.wait()
        pltpu.sync_copy(rows2, acc.at[idx_v.at[3]], add=True)
        return carry

    lax.fori_loop(0, NB2, body, 0)
    plsc.subcore_barrier()
    pltpu.sync_copy(acc.at[pl.ds(r0, RPS), :],
                    out_hbm.at[j, c, pl.ds(r0, RPS), :])


def _agg2_body(t0_hbm, t1_hbm, ea_hbm, eb_hbm, zeros_hbm, out_hbm,
               acc, idx_v, rows1, rows2, gs1, gs2, ss1, ss2):
    c = lax.axis_index("c")
    s = lax.axis_index("s")
    w = c * NS + s
    r0 = s * RPS
    for j, (t, e) in enumerate(((t0_hbm, ea_hbm), (t1_hbm, eb_hbm))):
        _agg_phase(t, e, zeros_hbm, out_hbm, j, acc,
                   idx_v, rows1, rows2, gs1, gs2, ss1, ss2, c, w, r0)
        plsc.subcore_barrier()


@functools.cache
def _agg2_kernel():
    return pl.kernel(
        _agg2_body,
        out_type=jax.ShapeDtypeStruct((2, NC, NP, IN), jnp.float32),
        mesh=_mesh(),
        scratch_types=[
            pltpu.VMEM_SHARED((NP, IN), jnp.float32),
            pltpu.VMEM((4, HB), jnp.int32),
            pltpu.VMEM((HB, IN), jnp.float32),
            pltpu.VMEM((HB, IN), jnp.float32),
            pltpu.SemaphoreType.DMA,
            pltpu.SemaphoreType.DMA,
            pltpu.SemaphoreType.DMA,
            pltpu.SemaphoreType.DMA,
        ],
    )


def _nrm(x):
    return jnp.where(x > 0, lax.rsqrt(x), 0.0)


def _prep_body(degp_ref, h_ref, xs0_ref, xs1_ref, norms_ref):
    d = degp_ref[0] + degp_ref[1]
    ns0 = _nrm(d[0])
    nd0 = _nrm(d[1])
    ns1 = _nrm(d[2])
    nd1 = _nrm(d[3])
    h = h_ref[...]
    xs0_ref[...] = h * ns0[:, :1]
    xs1_ref[...] = h * ns1[:, :1]
    norms_ref[...] = jnp.stack([ns0, nd0, ns1, nd1], axis=0)


def _branch_body(part_ref, norms_ref, w0_ref, b0_ref, w1_ref,
                 y0_ref, y1_ref, *, m):
    nd = norms_ref[2 * m + 1][:, :1]
    ns = norms_ref[2 * m][:, :1]
    agg = (part_ref[0, 0] + part_ref[0, 1]) * nd
    z = jnp.dot(agg, w0_ref[...], preferred_element_type=jnp.float32)
    z = jnp.maximum(z + b0_ref[...], 0.0)
    y = jnp.dot(z * ns, w1_ref[...], preferred_element_type=jnp.float32)
    y0_ref[...] = y[:, :IN]
    y1_ref[...] = y[:, IN:]


def _att_body(q0_ref, q1_ref, norms_ref, b01_ref, b11_ref, wp_ref, bp_ref,
              wqr_ref, e0_ref, e1_ref, sc_ref):
    i = pl.program_id(0)
    nd0 = norms_ref[1][:, :1]
    nd1 = norms_ref[3][:, :1]
    e0 = jnp.concatenate(
        [q0_ref[0, 0] + q0_ref[0, 1], q0_ref[1, 0] + q0_ref[1, 1]], axis=1)
    e0 = e0 * nd0 + b01_ref[...]
    e1 = jnp.concatenate(
        [q1_ref[0, 0] + q1_ref[0, 1], q1_ref[1, 0] + q1_ref[1, 1]], axis=1)
    e1 = e1 * nd1 + b11_ref[...]
    e0_ref[...] = e0
    e1_ref[...] = e1
    t0 = jnp.tanh(jnp.dot(e0, wp_ref[...],
                          preferred_element_type=jnp.float32) + bp_ref[...])
    t1 = jnp.tanh(jnp.dot(e1, wp_ref[...],
                          preferred_element_type=jnp.float32) + bp_ref[...])
    s0 = jnp.sum(t0 * wqr_ref[...])
    s1 = jnp.sum(t1 * wqr_ref[...])
    srow = jnp.concatenate(
        [jnp.full((1, 128), s0, jnp.float32),
         jnp.full((1, 128), s1, jnp.float32)], axis=0)

    @pl.when(i == 0)
    def _():
        sc_ref[...] = srow

    @pl.when(i > 0)
    def _():
        sc_ref[...] = sc_ref[...] + srow


def _mix_body(e0_ref, e1_ref, sc_ref, out_ref):
    w0 = jnp.max(sc_ref[0]) * (1.0 / N)
    w1 = jnp.max(sc_ref[1]) * (1.0 / N)
    mx = jnp.maximum(w0, w1)
    a = jnp.exp(w0 - mx)
    b = jnp.exp(w1 - mx)
    beta0 = a / (a + b)
    out_ref[...] = beta0 * e0_ref[...] + (1.0 - beta0) * e1_ref[...]


_prep_call = pl.pallas_call(
    _prep_body,
    grid=(GN,),
    in_specs=[
        pl.BlockSpec((NC, 4, RB, DL), lambda i: (0, 0, i, 0)),
        pl.BlockSpec((RB, IN), lambda i: (i, 0)),
    ],
    out_specs=[
        pl.BlockSpec((RB, IN), lambda i: (i, 0)),
        pl.BlockSpec((RB, IN), lambda i: (i, 0)),
        pl.BlockSpec((4, RB, DL), lambda i: (0, i, 0)),
    ],
    out_shape=[
        jax.ShapeDtypeStruct((NP, IN), jnp.float32),
        jax.ShapeDtypeStruct((NP, IN), jnp.float32),
        jax.ShapeDtypeStruct((4, N, DL), jnp.float32),
    ],
)


def _branch_call(m):
    return pl.pallas_call(
        functools.partial(_branch_body, m=m),
        grid=(GN,),
        in_specs=[
            pl.BlockSpec((1, NC, RB, IN), lambda i, m=m: (m, 0, i, 0)),
            pl.BlockSpec((4, RB, DL), lambda i: (0, i, 0)),
            pl.BlockSpec((IN, HIDP), lambda i: (0, 0)),
            pl.BlockSpec((1, HIDP), lambda i: (0, 0)),
            pl.BlockSpec((HIDP, OUT), lambda i: (0, 0)),
        ],
        out_specs=[
            pl.BlockSpec((RB, IN), lambda i: (i, 0)),
            pl.BlockSpec((RB, IN), lambda i: (i, 0)),
        ],
        out_shape=[
            jax.ShapeDtypeStruct((NP, IN), jnp.float32),
            jax.ShapeDtypeStruct((NP, IN), jnp.float32),
        ],
    )


_att_call = pl.pallas_call(
    _att_body,
    grid=(GN,),
    in_specs=[
        pl.BlockSpec((2, NC, RB, IN), lambda i: (0, 0, i, 0)),
        pl.BlockSpec((2, NC, RB, IN), lambda i: (0, 0, i, 0)),
        pl.BlockSpec((4, RB, DL), lambda i: (0, i, 0)),
        pl.BlockSpec((1, OUT), lambda i: (0, 0)),
        pl.BlockSpec((1, OUT), lambda i: (0, 0)),
        pl.BlockSpec((OUT, ATT_H), lambda i: (0, 0)),
        pl.BlockSpec((1, ATT_H), lambda i: (0, 0)),
        pl.BlockSpec((1, ATT_H), lambda i: (0, 0)),
    ],
    out_specs=[
        pl.BlockSpec((RB, OUT), lambda i: (i, 0)),
        pl.BlockSpec((RB, OUT), lambda i: (i, 0)),
        pl.BlockSpec((2, 128), lambda i: (0, 0)),
    ],
    out_shape=[
        jax.ShapeDtypeStruct((N, OUT), jnp.float32),
        jax.ShapeDtypeStruct((N, OUT), jnp.float32),
        jax.ShapeDtypeStruct((2, 128), jnp.float32),
    ],
)

_mix_call = pl.pallas_call(
    _mix_body,
    grid=(GN,),
    in_specs=[
        pl.BlockSpec((RB, OUT), lambda i: (i, 0)),
        pl.BlockSpec((RB, OUT), lambda i: (i, 0)),
        pl.BlockSpec((2, 128), lambda i: (0, 0)),
    ],
    out_specs=pl.BlockSpec((RB, OUT), lambda i: (i, 0)),
    out_shape=jax.ShapeDtypeStruct((N, OUT), jnp.float32),
)


def _pack_edges(src, dst):
    s = jnp.pad(src.reshape(NW, EPW), ((0, 0), (0, EPAD)), constant_values=N)
    d = jnp.pad(dst.reshape(NW, EPW), ((0, 0), (0, EPAD)), constant_values=N)
    s = s.reshape(NW, NB2, 2, HB)
    d = d.reshape(NW, NB2, 2, HB)
    return jnp.concatenate([s, d], axis=2)


def kernel(h, edge_index0, edge_index1,
           W00, b00, W01, b01, W10, b10, W11, b11, Wp, bp, Wq):
    f32 = jnp.float32
    e0 = _pack_edges(edge_index0[0], edge_index0[1])
    e1 = _pack_edges(edge_index1[0], edge_index1[1])
    z128 = jnp.zeros((NP, IN), f32)

    degp = _deg_kernel()(e0, e1)
    degp8 = jnp.broadcast_to(degp[..., None], (NC, 4, NP, DL))
    xs0, xs1, norms = _prep_call(degp8, h)

    p = _agg2_kernel()(xs0, xs1, e0, e1, z128)

    W0p0 = jnp.pad(W00, ((0, 0), (0, HIDP - HID)))
    b0p0 = jnp.pad(b00, (0, HIDP - HID)).reshape(1, HIDP)
    W1p0 = jnp.pad(W01, ((0, HIDP - HID), (0, 0)))
    W0p1 = jnp.pad(W10, ((0, 0), (0, HIDP - HID)))
    b0p1 = jnp.pad(b10, (0, HIDP - HID)).reshape(1, HIDP)
    W1p1 = jnp.pad(W11, ((0, HIDP - HID), (0, 0)))

    y00, y01 = _branch_call(0)(p, norms, W0p0, b0p0, W1p0)
    y10, y11 = _branch_call(1)(p, norms, W0p1, b0p1, W1p1)

    q0 = _agg2_kernel()(y00, y01, e0, e0, z128)
    q1 = _agg2_kernel()(y10, y11, e1, e1, z128)

    e0a, e1a, scores = _att_call(
        q0, q1, norms,
        b01.reshape(1, OUT), b11.reshape(1, OUT),
        Wp, bp.reshape(1, ATT_H), Wq.reshape(1, ATT_H))
    return _mix_call(e0a, e1a, scores)


# speedup vs baseline: 1.3795x; 1.3795x over previous
import functools

import jax
import jax.numpy as jnp
from jax import lax
from jax.experimental import pallas as pl
from jax.experimental.pallas import tpu as pltpu
from jax.experimental.pallas import tpu_sc as plsc

N = 10000
E = 320000
IN = 128
HID = 1000
HIDP = 1024
OUT = 256
ATT_H = 128

NP = 10240

NC = 2
NS = 16
NW = NC * NS
EPW = E // NW
HB = 64
EB = 2 * HB
NB2 = -(-EPW // EB)
EPAD = NB2 * EB - EPW
RPS = NP // NS
L = 16

DL = 8

RB = 400
GN = N // RB


@functools.cache
def _mesh():
    return plsc.VectorSubcoreMesh(
        core_axis_name="c", subcore_axis_name="s",
        num_cores=NC, num_subcores=NS)


def _agg_phase(table_hbm, eidx_hbm, zeros_hbm, out_hbm, j, acc,
               idx_v, rows1, rows2, sem1, sem2, c, w, r0):
    pltpu.sync_copy(zeros_hbm.at[pl.ds(r0, RPS), :],
                    acc.at[pl.ds(r0, RPS), :])
    plsc.subcore_barrier()

    def body(b, carry):
        pltpu.sync_copy(eidx_hbm.at[w, b], idx_v)
        cp1 = pltpu.async_copy(table_hbm.at[idx_v.at[0]], rows1, sem1)
        cp2 = pltpu.async_copy(table_hbm.at[idx_v.at[1]], rows2, sem2)
        cp1.wait()
        pltpu.sync_copy(rows1, acc.at[idx_v.at[2]], add=True)
        cp2.wait()
        pltpu.sync_copy(rows2, acc.at[idx_v.at[3]], add=True)
        return carry

    lax.fori_loop(0, NB2, body, 0)
    plsc.subcore_barrier()
    pltpu.sync_copy(acc.at[pl.ds(r0, RPS), :],
                    out_hbm.at[j, c, pl.ds(r0, RPS), :])


def _deg_body(e0_hbm, e1_hbm, degp_hbm,
              d0, d1, d2, d3, shared, idx_v, buf, accum):
    c = lax.axis_index("c")
    s = lax.axis_index("s")
    w = c * NS + s
    r0 = s * RPS
    degs = [d0, d1, d2, d3]
    zeros16 = jnp.zeros((L,), jnp.float32)
    ones16 = jnp.ones((L,), jnp.float32)

    def zbody(t, carry):
        for j in range(4):
            degs[j][pl.ds(t * L, L)] = zeros16
        return carry

    lax.fori_loop(0, NP // L, zbody, 0)

    for e_hbm, j0 in ((e0_hbm, 0), (e1_hbm, 2)):
        def body(b, carry, e_hbm=e_hbm, j0=j0):
            pltpu.sync_copy(e_hbm.at[w, b], idx_v)
            for r in range(4):
                dj = degs[j0 + (r // 2)]
                for i in range(HB // L):
                    idx = idx_v[r, pl.ds(i * L, L)]
                    plsc.addupdate_scatter(dj, [idx], ones16)
            return carry

        lax.fori_loop(0, NB2, body, 0)

    for j in range(4):
        pltpu.sync_copy(degs[j], shared.at[j, s])
    plsc.subcore_barrier()

    for j in range(4):
        pltpu.sync_copy(shared.at[j, :, pl.ds(r0, RPS)], buf)

        def abody(t, carry):
            acc = buf[0, pl.ds(t * L, L)]
            for k in range(1, NS):
                acc = acc + buf[k, pl.ds(t * L, L)]
            accum[pl.ds(t * L, L)] = acc
            return carry

        lax.fori_loop(0, RPS // L, abody, 0)
        pltpu.sync_copy(accum, degp_hbm.at[c, j, pl.ds(r0, RPS)])


@functools.cache
def _deg_kernel():
    return pl.kernel(
        _deg_body,
        out_type=jax.ShapeDtypeStruct((NC, 4, NP), jnp.float32),
        mesh=_mesh(),
        scratch_types=[
            pltpu.VMEM((NP,), jnp.float32),
            pltpu.VMEM((NP,), jnp.float32),
            pltpu.VMEM((NP,), jnp.float32),
            pltpu.VMEM((NP,), jnp.float32),
            pltpu.VMEM_SHARED((4, NS, NP), jnp.float32),
            pltpu.VMEM((4, HB), jnp.int32),
            pltpu.VMEM((NS, RPS), jnp.float32),
            pltpu.VMEM((RPS,), jnp.float32),
        ],
        compiler_params=pltpu.CompilerParams(needs_layout_passes=False),
    )


def _agg2_body(t0_hbm, t1_hbm, e0_hbm, e1_hbm, zeros_hbm, out_hbm,
               acc, idx_v, rows1, rows2, sem1, sem2):
    c = lax.axis_index("c")
    s = lax.axis_index("s")
    w = c * NS + s
    r0 = s * RPS
    for j, (t, e) in enumerate(((t0_hbm, e0_hbm), (t1_hbm, e1_hbm))):
        _agg_phase(t, e, zeros_hbm, out_hbm, j, acc,
                   idx_v, rows1, rows2, sem1, sem2, c, w, r0)
        plsc.subcore_barrier()


def _agg4_body(t0_hbm, t1_hbm, t2_hbm, t3_hbm, e0_hbm, e1_hbm, zeros_hbm,
               out_hbm, acc, idx_v, rows1, rows2, sem1, sem2):
    c = lax.axis_index("c")
    s = lax.axis_index("s")
    w = c * NS + s
    r0 = s * RPS
    for j, (t, e) in enumerate(((t0_hbm, e0_hbm), (t1_hbm, e0_hbm),
                                (t2_hbm, e1_hbm), (t3_hbm, e1_hbm))):
        _agg_phase(t, e, zeros_hbm, out_hbm, j, acc,
                   idx_v, rows1, rows2, sem1, sem2, c, w, r0)
        plsc.subcore_barrier()


def _agg_scratch():
    return [
        pltpu.VMEM_SHARED((NP, IN), jnp.float32),
        pltpu.VMEM((4, HB), jnp.int32),
        pltpu.VMEM((HB, IN), jnp.float32),
        pltpu.VMEM((HB, IN), jnp.float32),
        pltpu.SemaphoreType.DMA,
        pltpu.SemaphoreType.DMA,
    ]


@functools.cache
def _agg2_kernel():
    return pl.kernel(
        _agg2_body,
        out_type=jax.ShapeDtypeStruct((2, NC, NP, IN), jnp.float32),
        mesh=_mesh(),
        scratch_types=_agg_scratch(),
    )


@functools.cache
def _agg4_kernel():
    return pl.kernel(
        _agg4_body,
        out_type=jax.ShapeDtypeStruct((4, NC, NP, IN), jnp.float32),
        mesh=_mesh(),
        scratch_types=_agg_scratch(),
    )


def _nrm(x):
    return jnp.where(x > 0, lax.rsqrt(x), 0.0)


def _prep_body(degp_ref, h_ref, xs0_ref, xs1_ref, norms_ref):
    d = degp_ref[0] + degp_ref[1]
    ns0 = _nrm(d[0])
    nd0 = _nrm(d[1])
    ns1 = _nrm(d[2])
    nd1 = _nrm(d[3])
    h = h_ref[...]
    xs0_ref[...] = h * ns0[:, :1]
    xs1_ref[...] = h * ns1[:, :1]
    norms_ref[...] = jnp.stack([ns0, nd0, ns1, nd1], axis=0)


def _branch_body(part_ref, norms_ref, w0_ref, b0_ref, w1_ref,
                 y0_ref, y1_ref, *, m):
    nd = norms_ref[2 * m + 1][:, :1]
    ns = norms_ref[2 * m][:, :1]
    agg = (part_ref[0, 0] + part_ref[0, 1]) * nd
    z = jnp.dot(agg, w0_ref[...], preferred_element_type=jnp.float32)
    z = jnp.maximum(z + b0_ref[...], 0.0)
    y = jnp.dot(z * ns, w1_ref[...], preferred_element_type=jnp.float32)
    y0_ref[...] = y[:, :IN]
    y1_ref[...] = y[:, IN:]


def _att_body(q_ref, norms_ref, b01_ref, b11_ref, wp_ref, bp_ref, wqr_ref,
              e0_ref, e1_ref, sc_ref):
    i = pl.program_id(0)
    nd0 = norms_ref[1][:, :1]
    nd1 = norms_ref[3][:, :1]
    e0 = jnp.concatenate(
        [q_ref[0, 0] + q_ref[0, 1], q_ref[1, 0] + q_ref[1, 1]], axis=1)
    e0 = e0 * nd0 + b01_ref[...]
    e1 = jnp.concatenate(
        [q_ref[2, 0] + q_ref[2, 1], q_ref[3, 0] + q_ref[3, 1]], axis=1)
    e1 = e1 * nd1 + b11_ref[...]
    e0_ref[...] = e0
    e1_ref[...] = e1
    t0 = jnp.tanh(jnp.dot(e0, wp_ref[...],
                          preferred_element_type=jnp.float32) + bp_ref[...])
    t1 = jnp.tanh(jnp.dot(e1, wp_ref[...],
                          preferred_element_type=jnp.float32) + bp_ref[...])
    s0 = jnp.sum(t0 * wqr_ref[...])
    s1 = jnp.sum(t1 * wqr_ref[...])
    srow = jnp.concatenate(
        [jnp.full((1, 128), s0, jnp.float32),
         jnp.full((1, 128), s1, jnp.float32)], axis=0)

    @pl.when(i == 0)
    def _():
        sc_ref[...] = srow

    @pl.when(i > 0)
    def _():
        sc_ref[...] = sc_ref[...] + srow


def _mix_body(e0_ref, e1_ref, sc_ref, out_ref):
    w0 = jnp.max(sc_ref[0]) * (1.0 / N)
    w1 = jnp.max(sc_ref[1]) * (1.0 / N)
    mx = jnp.maximum(w0, w1)
    a = jnp.exp(w0 - mx)
    b = jnp.exp(w1 - mx)
    beta0 = a / (a + b)
    out_ref[...] = beta0 * e0_ref[...] + (1.0 - beta0) * e1_ref[...]


_prep_call = pl.pallas_call(
    _prep_body,
    grid=(GN,),
    in_specs=[
        pl.BlockSpec((NC, 4, RB, DL), lambda i: (0, 0, i, 0)),
        pl.BlockSpec((RB, IN), lambda i: (i, 0)),
    ],
    out_specs=[
        pl.BlockSpec((RB, IN), lambda i: (i, 0)),
        pl.BlockSpec((RB, IN), lambda i: (i, 0)),
        pl.BlockSpec((4, RB, DL), lambda i: (0, i, 0)),
    ],
    out_shape=[
        jax.ShapeDtypeStruct((NP, IN), jnp.float32),
        jax.ShapeDtypeStruct((NP, IN), jnp.float32),
        jax.ShapeDtypeStruct((4, N, DL), jnp.float32),
    ],
)


def _branch_call(m):
    return pl.pallas_call(
        functools.partial(_branch_body, m=m),
        grid=(GN,),
        in_specs=[
            pl.BlockSpec((1, NC, RB, IN), lambda i, m=m: (m, 0, i, 0)),
            pl.BlockSpec((4, RB, DL), lambda i: (0, i, 0)),
            pl.BlockSpec((IN, HIDP), lambda i: (0, 0)),
            pl.BlockSpec((1, HIDP), lambda i: (0, 0)),
            pl.BlockSpec((HIDP, OUT), lambda i: (0, 0)),
        ],
        out_specs=[
            pl.BlockSpec((RB, IN), lambda i: (i, 0)),
            pl.BlockSpec((RB, IN), lambda i: (i, 0)),
        ],
        out_shape=[
            jax.ShapeDtypeStruct((NP, IN), jnp.float32),
            jax.ShapeDtypeStruct((NP, IN), jnp.float32),
        ],
    )


_att_call = pl.pallas_call(
    _att_body,
    grid=(GN,),
    in_specs=[
        pl.BlockSpec((4, NC, RB, IN), lambda i: (0, 0, i, 0)),
        pl.BlockSpec((4, RB, DL), lambda i: (0, i, 0)),
        pl.BlockSpec((1, OUT), lambda i: (0, 0)),
        pl.BlockSpec((1, OUT), lambda i: (0, 0)),
        pl.BlockSpec((OUT, ATT_H), lambda i: (0, 0)),
        pl.BlockSpec((1, ATT_H), lambda i: (0, 0)),
        pl.BlockSpec((1, ATT_H), lambda i: (0, 0)),
    ],
    out_specs=[
        pl.BlockSpec((RB, OUT), lambda i: (i, 0)),
        pl.BlockSpec((RB, OUT), lambda i: (i, 0)),
        pl.BlockSpec((2, 128), lambda i: (0, 0)),
    ],
    out_shape=[
        jax.ShapeDtypeStruct((N, OUT), jnp.float32),
        jax.ShapeDtypeStruct((N, OUT), jnp.float32),
        jax.ShapeDtypeStruct((2, 128), jnp.float32),
    ],
)

_mix_call = pl.pallas_call(
    _mix_body,
    grid=(GN,),
    in_specs=[
        pl.BlockSpec((RB, OUT), lambda i: (i, 0)),
        pl.BlockSpec((RB, OUT), lambda i: (i, 0)),
        pl.BlockSpec((2, 128), lambda i: (0, 0)),
    ],
    out_specs=pl.BlockSpec((RB, OUT), lambda i: (i, 0)),
    out_shape=jax.ShapeDtypeStruct((N, OUT), jnp.float32),
)


def _pack_edges(src, dst):
    s = jnp.pad(src.reshape(NW, EPW), ((0, 0), (0, EPAD)), constant_values=N)
    d = jnp.pad(dst.reshape(NW, EPW), ((0, 0), (0, EPAD)), constant_values=N)
    s = s.reshape(NW, NB2, 2, HB)
    d = d.reshape(NW, NB2, 2, HB)
    return jnp.concatenate([s, d], axis=2)


def kernel(h, edge_index0, edge_index1,
           W00, b00, W01, b01, W10, b10, W11, b11, Wp, bp, Wq):
    f32 = jnp.float32
    e0 = _pack_edges(edge_index0[0], edge_index0[1])
    e1 = _pack_edges(edge_index1[0], edge_index1[1])
    z128 = jnp.zeros((NP, IN), f32)

    degp = _deg_kernel()(e0, e1)
    degp8 = jnp.broadcast_to(degp[..., None], (NC, 4, NP, DL))
    xs0, xs1, norms = _prep_call(degp8, h)

    p = _agg2_kernel()(xs0, xs1, e0, e1, z128)

    W0p0 = jnp.pad(W00, ((0, 0), (0, HIDP - HID)))
    b0p0 = jnp.pad(b00, (0, HIDP - HID)).reshape(1, HIDP)
    W1p0 = jnp.pad(W01, ((0, HIDP - HID), (0, 0)))
    W0p1 = jnp.pad(W10, ((0, 0), (0, HIDP - HID)))
    b0p1 = jnp.pad(b10, (0, HIDP - HID)).reshape(1, HIDP)
    W1p1 = jnp.pad(W11, ((0, HIDP - HID), (0, 0)))

    y00, y01 = _branch_call(0)(p, norms, W0p0, b0p0, W1p0)
    y10, y11 = _branch_call(1)(p, norms, W0p1, b0p1, W1p1)

    q = _agg4_kernel()(y00, y01, y10, y11, e0, e1, z128)

    e0a, e1a, scores = _att_call(
        q, norms,
        b01.reshape(1, OUT), b11.reshape(1, OUT),
        Wp, bp.reshape(1, ATT_H), Wq.reshape(1, ATT_H))
    return _mix_call(e0a, e1a, scores)


# speedup vs baseline: 1.3844x; 1.0036x over previous
import functools

import jax
import jax.numpy as jnp
from jax import lax
from jax.experimental import pallas as pl
from jax.experimental.pallas import tpu as pltpu
from jax.experimental.pallas import tpu_sc as plsc

N = 10000
E = 320000
IN = 128
HID = 1000
HIDP = 1024
OUT = 256
ATT_H = 128

NP = 10240

NC = 2
NS = 16
NW = NC * NS
EPW = E // NW
HB = 64
EB = 2 * HB
NB2 = -(-EPW // EB)
EPAD = NB2 * EB - EPW
RPS = NP // NS
L = 16

DL = 8

RB = 400
GN = N // RB


@functools.cache
def _mesh():
    return plsc.VectorSubcoreMesh(
        core_axis_name="c", subcore_axis_name="s",
        num_cores=NC, num_subcores=NS)


def _agg_phase(table_hbm, eidx_hbm, zeros_hbm, out_hbm, j, acc,
               idx_v, rows1, rows2, sem1, sem2, c, w, r0):
    pltpu.sync_copy(zeros_hbm.at[pl.ds(r0, RPS), :],
                    acc.at[pl.ds(r0, RPS), :])
    plsc.subcore_barrier()

    def body(b, carry):
        pltpu.sync_copy(eidx_hbm.at[w, b], idx_v)
        cp1 = pltpu.async_copy(table_hbm.at[idx_v.at[0]], rows1, sem1)
        cp2 = pltpu.async_copy(table_hbm.at[idx_v.at[1]], rows2, sem2)
        cp1.wait()
        pltpu.sync_copy(rows1, acc.at[idx_v.at[2]], add=True)
        cp2.wait()
        pltpu.sync_copy(rows2, acc.at[idx_v.at[3]], add=True)
        return carry

    lax.fori_loop(0, NB2, body, 0)
    plsc.subcore_barrier()
    pltpu.sync_copy(acc.at[pl.ds(r0, RPS), :],
                    out_hbm.at[j, c, pl.ds(r0, RPS), :])


def _deg_body(e0_hbm, e1_hbm, degp_hbm,
              d0, d1, d2, d3, shared, idx_v, buf, accum):
    c = lax.axis_index("c")
    s = lax.axis_index("s")
    w = c * NS + s
    r0 = s * RPS
    degs = [d0, d1, d2, d3]
    zeros16 = jnp.zeros((L,), jnp.float32)
    ones16 = jnp.ones((L,), jnp.float32)

    def zbody(t, carry):
        for j in range(4):
            degs[j][pl.ds(t * L, L)] = zeros16
        return carry

    lax.fori_loop(0, NP // L, zbody, 0)

    for e_hbm, j0 in ((e0_hbm, 0), (e1_hbm, 2)):
        def body(b, carry, e_hbm=e_hbm, j0=j0):
            pltpu.sync_copy(e_hbm.at[w, b], idx_v)
            for r in range(4):
                dj = degs[j0 + (r // 2)]
                for i in range(HB // L):
                    idx = idx_v[r, pl.ds(i * L, L)]
                    plsc.addupdate_scatter(dj, [idx], ones16)
            return carry

        lax.fori_loop(0, NB2, body, 0)

    for j in range(4):
        pltpu.sync_copy(degs[j], shared.at[j, s])
    plsc.subcore_barrier()

    for j in range(4):
        pltpu.sync_copy(shared.at[j, :, pl.ds(r0, RPS)], buf)

        def abody(t, carry):
            acc = buf[0, pl.ds(t * L, L)]
            for k in range(1, NS):
                acc = acc + buf[k, pl.ds(t * L, L)]
            accum[pl.ds(t * L, L)] = acc
            return carry

        lax.fori_loop(0, RPS // L, abody, 0)
        pltpu.sync_copy(accum, degp_hbm.at[c, j, pl.ds(r0, RPS)])


@functools.cache
def _deg_kernel():
    return pl.kernel(
        _deg_body,
        out_type=jax.ShapeDtypeStruct((NC, 4, NP), jnp.float32),
        mesh=_mesh(),
        scratch_types=[
            pltpu.VMEM((NP,), jnp.float32),
            pltpu.VMEM((NP,), jnp.float32),
            pltpu.VMEM((NP,), jnp.float32),
            pltpu.VMEM((NP,), jnp.float32),
            pltpu.VMEM_SHARED((4, NS, NP), jnp.float32),
            pltpu.VMEM((4, HB), jnp.int32),
            pltpu.VMEM((NS, RPS), jnp.float32),
            pltpu.VMEM((RPS,), jnp.float32),
        ],
        compiler_params=pltpu.CompilerParams(needs_layout_passes=False),
    )


def _agg2_body(t0_hbm, t1_hbm, e0_hbm, e1_hbm, zeros_hbm, out_hbm,
               acc, idx_v, rows1, rows2, sem1, sem2):
    c = lax.axis_index("c")
    s = lax.axis_index("s")
    w = c * NS + s
    r0 = s * RPS
    for j, (t, e) in enumerate(((t0_hbm, e0_hbm), (t1_hbm, e1_hbm))):
        _agg_phase(t, e, zeros_hbm, out_hbm, j, acc,
                   idx_v, rows1, rows2, sem1, sem2, c, w, r0)
        plsc.subcore_barrier()


def _agg4_body(t0_hbm, t1_hbm, t2_hbm, t3_hbm, e0_hbm, e1_hbm, zeros_hbm,
               out_hbm, acc, idx_v, rows1, rows2, sem1, sem2):
    c = lax.axis_index("c")
    s = lax.axis_index("s")
    w = c * NS + s
    r0 = s * RPS
    for j, (t, e) in enumerate(((t0_hbm, e0_hbm), (t1_hbm, e0_hbm),
                                (t2_hbm, e1_hbm), (t3_hbm, e1_hbm))):
        _agg_phase(t, e, zeros_hbm, out_hbm, j, acc,
                   idx_v, rows1, rows2, sem1, sem2, c, w, r0)
        plsc.subcore_barrier()


def _agg_scratch():
    return [
        pltpu.VMEM_SHARED((NP, IN), jnp.float32),
        pltpu.VMEM((4, HB), jnp.int32),
        pltpu.VMEM((HB, IN), jnp.float32),
        pltpu.VMEM((HB, IN), jnp.float32),
        pltpu.SemaphoreType.DMA,
        pltpu.SemaphoreType.DMA,
    ]


@functools.cache
def _agg2_kernel():
    return pl.kernel(
        _agg2_body,
        out_type=jax.ShapeDtypeStruct((2, NC, NP, IN), jnp.float32),
        mesh=_mesh(),
        scratch_types=_agg_scratch(),
    )


@functools.cache
def _agg4_kernel():
    return pl.kernel(
        _agg4_body,
        out_type=jax.ShapeDtypeStruct((4, NC, NP, IN), jnp.float32),
        mesh=_mesh(),
        scratch_types=_agg_scratch(),
    )


def _nrm(x):
    return jnp.where(x > 0, lax.rsqrt(x), 0.0)


def _prep_body(degp_ref, h_ref, xs0_ref, xs1_ref, norms_ref):
    d = degp_ref[0] + degp_ref[1]
    ns0 = _nrm(d[0])
    nd0 = _nrm(d[1])
    ns1 = _nrm(d[2])
    nd1 = _nrm(d[3])
    h = h_ref[...]
    xs0_ref[...] = h * ns0[:, :1]
    xs1_ref[...] = h * ns1[:, :1]
    norms_ref[...] = jnp.stack([ns0, nd0, ns1, nd1], axis=0)


def _branch_body(part_ref, norms_ref, w0_ref, b0_ref, w1_ref,
                 y0_ref, y1_ref, *, m):
    nd = norms_ref[2 * m + 1][:, :1]
    ns = norms_ref[2 * m][:, :1]
    agg = (part_ref[0, 0] + part_ref[0, 1]) * nd
    z = jnp.dot(agg, w0_ref[...], preferred_element_type=jnp.float32)
    z = jnp.maximum(z + b0_ref[...], 0.0)
    y = jnp.dot(z * ns, w1_ref[...], preferred_element_type=jnp.float32)
    y0_ref[...] = y[:, :IN]
    y1_ref[...] = y[:, IN:]


def _att_body(q_ref, norms_ref, b01_ref, b11_ref, wp_ref, bp_ref, wqr_ref,
              e0_ref, e1_ref, sc_ref):
    i = pl.program_id(0)
    nd0 = norms_ref[1][:, :1]
    nd1 = norms_ref[3][:, :1]
    e0 = jnp.concatenate(
        [q_ref[0, 0] + q_ref[0, 1], q_ref[1, 0] + q_ref[1, 1]], axis=1)
    e0 = e0 * nd0 + b01_ref[...]
    e1 = jnp.concatenate(
        [q_ref[2, 0] + q_ref[2, 1], q_ref[3, 0] + q_ref[3, 1]], axis=1)
    e1 = e1 * nd1 + b11_ref[...]
    e0_ref[...] = e0
    e1_ref[...] = e1
    t0 = jnp.tanh(jnp.dot(e0, wp_ref[...],
                          preferred_element_type=jnp.float32) + bp_ref[...])
    t1 = jnp.tanh(jnp.dot(e1, wp_ref[...],
                          preferred_element_type=jnp.float32) + bp_ref[...])
    s0 = jnp.sum(t0 * wqr_ref[...])
    s1 = jnp.sum(t1 * wqr_ref[...])
    srow = jnp.concatenate(
        [jnp.full((1, 128), s0, jnp.float32),
         jnp.full((1, 128), s1, jnp.float32)], axis=0)

    @pl.when(i == 0)
    def _():
        sc_ref[...] = srow

    @pl.when(i > 0)
    def _():
        sc_ref[...] = sc_ref[...] + srow


def _mix_body(e0_ref, e1_ref, sc_ref, out_ref):
    w0 = jnp.max(sc_ref[0]) * (1.0 / N)
    w1 = jnp.max(sc_ref[1]) * (1.0 / N)
    mx = jnp.maximum(w0, w1)
    a = jnp.exp(w0 - mx)
    b = jnp.exp(w1 - mx)
    beta0 = a / (a + b)
    out_ref[...] = beta0 * e0_ref[...] + (1.0 - beta0) * e1_ref[...]


_prep_call = pl.pallas_call(
    _prep_body,
    grid=(GN,),
    in_specs=[
        pl.BlockSpec((NC, 4, RB, DL), lambda i: (0, 0, i, 0)),
        pl.BlockSpec((RB, IN), lambda i: (i, 0)),
    ],
    out_specs=[
        pl.BlockSpec((RB, IN), lambda i: (i, 0)),
        pl.BlockSpec((RB, IN), lambda i: (i, 0)),
        pl.BlockSpec((4, RB, DL), lambda i: (0, i, 0)),
    ],
    out_shape=[
        jax.ShapeDtypeStruct((NP, IN), jnp.float32),
        jax.ShapeDtypeStruct((NP, IN), jnp.float32),
        jax.ShapeDtypeStruct((4, N, DL), jnp.float32),
    ],
)


def _branch_call(m):
    return pl.pallas_call(
        functools.partial(_branch_body, m=m),
        grid=(GN,),
        in_specs=[
            pl.BlockSpec((1, NC, RB, IN), lambda i, m=m: (m, 0, i, 0)),
            pl.BlockSpec((4, RB, DL), lambda i: (0, i, 0)),
            pl.BlockSpec((IN, HIDP), lambda i: (0, 0)),
            pl.BlockSpec((1, HIDP), lambda i: (0, 0)),
            pl.BlockSpec((HIDP, OUT), lambda i: (0, 0)),
        ],
        out_specs=[
            pl.BlockSpec((RB, IN), lambda i: (i, 0)),
            pl.BlockSpec((RB, IN), lambda i: (i, 0)),
        ],
        out_shape=[
            jax.ShapeDtypeStruct((NP, IN), jnp.float32),
            jax.ShapeDtypeStruct((NP, IN), jnp.float32),
        ],
    )


_att_call = pl.pallas_call(
    _att_body,
    grid=(GN,),
    in_specs=[
        pl.BlockSpec((4, NC, RB, IN), lambda i: (0, 0, i, 0)),
        pl.BlockSpec((4, RB, DL), lambda i: (0, i, 0)),
        pl.BlockSpec((1, OUT), lambda i: (0, 0)),
        pl.BlockSpec((1, OUT), lambda i: (0, 0)),
        pl.BlockSpec((OUT, ATT_H), lambda i: (0, 0)),
        pl.BlockSpec((1, ATT_H), lambda i: (0, 0)),
        pl.BlockSpec((1, ATT_H), lambda i: (0, 0)),
    ],
    out_specs=[
        pl.BlockSpec((RB, OUT), lambda i: (i, 0)),
        pl.BlockSpec((RB, OUT), lambda i: (i, 0)),
        pl.BlockSpec((2, 128), lambda i: (0, 0)),
    ],
    out_shape=[
        jax.ShapeDtypeStruct((N, OUT), jnp.float32),
        jax.ShapeDtypeStruct((N, OUT), jnp.float32),
        jax.ShapeDtypeStruct((2, 128), jnp.float32),
    ],
)

_mix_call = pl.pallas_call(
    _mix_body,
    grid=(GN,),
    in_specs=[
        pl.BlockSpec((RB, OUT), lambda i: (i, 0)),
        pl.BlockSpec((RB, OUT), lambda i: (i, 0)),
        pl.BlockSpec((2, 128), lambda i: (0, 0)),
    ],
    out_specs=pl.BlockSpec((RB, OUT), lambda i: (i, 0)),
    out_shape=jax.ShapeDtypeStruct((N, OUT), jnp.float32),
)


def _pack_edges(src, dst):
    s = jnp.pad(src.reshape(NW, EPW), ((0, 0), (0, EPAD)), constant_values=N)
    w = jnp.arange(NW, dtype=jnp.int32)[:, None]
    k = jnp.arange(EPAD, dtype=jnp.int32)[None, :]
    pad_rows = N + (w * EPAD + k) % (NP - N)
    d = jnp.concatenate(
        [dst.reshape(NW, EPW), pad_rows.astype(jnp.int32)], axis=1)
    s = s.reshape(NW, NB2, 2, HB)
    d = d.reshape(NW, NB2, 2, HB)
    return jnp.concatenate([s, d], axis=2)


def kernel(h, edge_index0, edge_index1,
           W00, b00, W01, b01, W10, b10, W11, b11, Wp, bp, Wq):
    f32 = jnp.float32
    e0 = _pack_edges(edge_index0[0], edge_index0[1])
    e1 = _pack_edges(edge_index1[0], edge_index1[1])
    z128 = jnp.zeros((NP, IN), f32)

    degp = _deg_kernel()(e0, e1)
    degp8 = jnp.broadcast_to(degp[..., None], (NC, 4, NP, DL))
    xs0, xs1, norms = _prep_call(degp8, h)

    p = _agg2_kernel()(xs0, xs1, e0, e1, z128)

    W0p0 = jnp.pad(W00, ((0, 0), (0, HIDP - HID)))
    b0p0 = jnp.pad(b00, (0, HIDP - HID)).reshape(1, HIDP)
    W1p0 = jnp.pad(W01, ((0, HIDP - HID), (0, 0)))
    W0p1 = jnp.pad(W10, ((0, 0), (0, HIDP - HID)))
    b0p1 = jnp.pad(b10, (0, HIDP - HID)).reshape(1, HIDP)
    W1p1 = jnp.pad(W11, ((0, HIDP - HID), (0, 0)))

    y00, y01 = _branch_call(0)(p, norms, W0p0, b0p0, W1p0)
    y10, y11 = _branch_call(1)(p, norms, W0p1, b0p1, W1p1)

    q = _agg4_kernel()(y00, y01, y10, y11, e0, e1, z128)

    e0a, e1a, scores = _att_call(
        q, norms,
        b01.reshape(1, OUT), b11.reshape(1, OUT),
        Wp, bp.reshape(1, ATT_H), Wq.reshape(1, ATT_H))
    return _mix_call(e0a, e1a, scores)
